# Initial kernel scaffold; baseline (speedup 1.0000x reference)
import jax
import jax.numpy as jnp
from jax import lax
from jax.experimental import pallas as pl
from jax.experimental.pallas import tpu as pltpu
from jax.experimental.pallas import tpu_sc as plsc

N = 100000
E = 3200000
H = 16

NC = 2
NS = 16
L = 16

N_PAD = 102400
CHUNK = 4096
G = CHUNK // 128
N_TILES = NC * NS
CHUNKS_PER_TILE = 25
E_PAD = N_TILES * CHUNKS_PER_TILE * CHUNK
ACC_SLICE = N_PAD // NS


def _edge_pass(table_hbm, src_hbm, w_hbm, dst_hbm, out_hbm,
               acc_sh, table_v, src_v, w_v, m_v, dst_v, slice_v, dma_sem):
    c = lax.axis_index("c")
    s = lax.axis_index("s")
    wid = c * NS + s

    pltpu.sync_copy(table_hbm, table_v)

    def zero_body(i, _):
        slice_v[pl.ds(i * L, L)] = jnp.zeros((L,), jnp.float32)
        return 0
    lax.fori_loop(0, ACC_SLICE // L, zero_body, 0)
    pltpu.sync_copy(slice_v, acc_sh.at[pl.ds(s * ACC_SLICE, ACC_SLICE)])
    plsc.subcore_barrier()

    def chunk_body(i, _):
        row = (wid * CHUNKS_PER_TILE + i) * G
        pltpu.sync_copy(src_hbm.at[pl.ds(row, G)], src_v)
        pltpu.sync_copy(w_hbm.at[pl.ds(row, G)], w_v)
        pltpu.sync_copy(dst_hbm.at[pl.ds(row, G)], dst_v)

        def gather_body(j, _):
            for k in range(128 // L):
                idx = src_v[j, pl.ds(k * L, L)]
                vals = plsc.load_gather(table_v, [idx])
                m_v[j, pl.ds(k * L, L)] = vals * w_v[j, pl.ds(k * L, L)]
            return 0
        lax.fori_loop(0, G, gather_body, 0)

        handles = []
        for j in range(G):
            handles.append(
                pltpu.async_copy(m_v.at[j], acc_sh.at[dst_v.at[j]],
                                 dma_sem, add=True))
        for h in handles:
            h.wait()
        return 0

    lax.fori_loop(0, CHUNKS_PER_TILE, chunk_body, 0)
    plsc.subcore_barrier()

    pltpu.sync_copy(acc_sh.at[pl.ds(s * ACC_SLICE, ACC_SLICE)], slice_v)
    pltpu.sync_copy(slice_v, out_hbm.at[c, pl.ds(s * ACC_SLICE, ACC_SLICE)])


def _make_edge_pass(interpret=False):
    mesh = plsc.VectorSubcoreMesh(core_axis_name="c", subcore_axis_name="s",
                                  num_cores=NC, num_subcores=NS)
    return pl.kernel(
        _edge_pass,
        out_type=jax.ShapeDtypeStruct((NC, N_PAD), jnp.float32),
        mesh=mesh,
        scratch_types=[
            pltpu.VMEM_SHARED((N_PAD,), jnp.float32),
            pltpu.VMEM((N_PAD,), jnp.float32),
            pltpu.VMEM((G, 128), jnp.int32),
            pltpu.VMEM((G, 128), jnp.float32),
            pltpu.VMEM((G, 128), jnp.float32),
            pltpu.VMEM((G, 128), jnp.int32),
            pltpu.VMEM((ACC_SLICE,), jnp.float32),
            pltpu.SemaphoreType.DMA,
        ],
        interpret=interpret,
    )


def _node_stage(w_ref, s_ref, x_ref, t_ref, r_ref):
    sv = s_ref[...]
    xv = x_ref[...]
    t = jnp.zeros_like(sv)
    r = jnp.zeros_like(sv)
    for k in range(H):
        h = jnp.maximum(sv * w_ref[0, k] + xv * w_ref[2, k] + w_ref[1, k], 0.0)
        t = t + h * w_ref[3, k]
        r = r + h * w_ref[4, k]
    t_ref[...] = t
    r_ref[...] = r + w_ref[5, 0]


_ROWS = N_PAD // 128
_BLK = 80


def _node_kernel(wmat, s2d, x2d, interpret=False):
    return pl.pallas_call(
        _node_stage,
        grid=(_ROWS // _BLK,),
        in_specs=[
            pl.BlockSpec(memory_space=pltpu.SMEM),
            pl.BlockSpec((_BLK, 128), lambda i: (i, 0)),
            pl.BlockSpec((_BLK, 128), lambda i: (i, 0)),
        ],
        out_specs=[
            pl.BlockSpec((_BLK, 128), lambda i: (i, 0)),
            pl.BlockSpec((_BLK, 128), lambda i: (i, 0)),
        ],
        out_shape=[
            jax.ShapeDtypeStruct((_ROWS, 128), jnp.float32),
            jax.ShapeDtypeStruct((_ROWS, 128), jnp.float32),
        ],
        interpret=interpret,
    )(wmat, s2d, x2d)


@jax.jit
def _run(x, edge_index, edge_weight,
         W1_rel, b1_rel, W1_root, W2_rel, b2_rel, W2_root):
    src = edge_index[0]
    dst = edge_index[1]
    pad = E_PAD - E
    src_p = jnp.concatenate([src, jnp.zeros((pad,), jnp.int32)])
    dst_p = jnp.concatenate([dst, jnp.full((pad,), N, jnp.int32)])
    w_p = jnp.concatenate([edge_weight, jnp.zeros((pad,), jnp.float32)])
    src2d = src_p.reshape(E_PAD // 128, 128)
    dst2d = dst_p.reshape(E_PAD // 128, 128)
    w2d = w_p.reshape(E_PAD // 128, 128)

    x_flat = x.reshape(-1)
    x_pad = jnp.concatenate([x_flat, jnp.zeros((N_PAD - N,), jnp.float32)])

    edge_pass = _make_edge_pass()

    s_part = edge_pass(x_pad, src2d, w2d, dst2d)
    s_pad = s_part[0] + s_part[1]

    wmat = jnp.stack([
        W1_rel[:, 0], b1_rel, W1_root[:, 0],
        W2_rel[0, :], W2_root[0, :],
        jnp.full((H,), b2_rel[0], jnp.float32),
    ])
    t2d, r2d = _node_kernel(wmat, s_pad.reshape(_ROWS, 128),
                            x_pad.reshape(_ROWS, 128))
    t_pad = t2d.reshape(-1)

    u_part = edge_pass(t_pad, src2d, w2d, dst2d)
    out_pad = u_part[0] + u_part[1] + r2d.reshape(-1)
    return out_pad[:N]


def kernel(x, edge_index, edge_weight,
           W1_rel, b1_rel, W1_root, W2_rel, b2_rel, W2_root):
    return _run(x, edge_index, edge_weight,
                W1_rel, b1_rel, W1_root, W2_rel, b2_rel, W2_root)


# speedup vs baseline: 118.9252x; 118.9252x over previous
import jax
import jax.numpy as jnp
from jax import lax
from jax.experimental import pallas as pl
from jax.experimental.pallas import tpu as pltpu
from jax.experimental.pallas import tpu_sc as plsc

N = 100000
E = 3200000
H = 16

NC = 2
NS = 16
L = 16

N_PAD = 102400
CHUNK = 4096
G = CHUNK // 128
N_TILES = NC * NS
CHUNKS_PER_TILE = 25
E_PAD = N_TILES * CHUNKS_PER_TILE * CHUNK
ACC_SLICE = N_PAD // NS


def _edge_pass(table_hbm, src_hbm, w_hbm, dst_hbm, out_hbm,
               acc_sh, table_v, src_v, w_v, m_v, dst_v, slice_v, dma_sem):
    c = lax.axis_index("c")
    s = lax.axis_index("s")
    wid = c * NS + s

    pltpu.sync_copy(table_hbm, table_v)

    def zero_body(i, _):
        slice_v[pl.ds(i * L, L)] = jnp.zeros((L,), jnp.float32)
        return 0
    lax.fori_loop(0, CHUNK // L, zero_body, 0)
    base = s * ACC_SLICE
    pltpu.sync_copy(slice_v, acc_sh.at[pl.ds(base, CHUNK)])
    pltpu.sync_copy(slice_v.at[pl.ds(0, ACC_SLICE - CHUNK)],
                    acc_sh.at[pl.ds(base + CHUNK, ACC_SLICE - CHUNK)])
    plsc.subcore_barrier()

    def chunk_body(i, _):
        row = (wid * CHUNKS_PER_TILE + i) * G
        pltpu.sync_copy(src_hbm.at[pl.ds(row, G)], src_v)
        pltpu.sync_copy(w_hbm.at[pl.ds(row, G)], w_v)
        pltpu.sync_copy(dst_hbm.at[pl.ds(row, G)], dst_v)

        def gather_body(j, _):
            for k in range(128 // L):
                idx = src_v[j, pl.ds(k * L, L)]
                vals = plsc.load_gather(table_v, [idx])
                m_v[j, pl.ds(k * L, L)] = vals * w_v[j, pl.ds(k * L, L)]
            return 0
        lax.fori_loop(0, G, gather_body, 0)

        handles = []
        for j in range(G):
            handles.append(
                pltpu.async_copy(m_v.at[j], acc_sh.at[dst_v.at[j]],
                                 dma_sem, add=True))
        for h in handles:
            h.wait()
        return 0

    lax.fori_loop(0, CHUNKS_PER_TILE, chunk_body, 0)
    plsc.subcore_barrier()

    pltpu.sync_copy(acc_sh.at[pl.ds(base, ACC_SLICE)],
                    out_hbm.at[c, pl.ds(base, ACC_SLICE)])


def _make_edge_pass(interpret=False):
    mesh = plsc.VectorSubcoreMesh(core_axis_name="c", subcore_axis_name="s",
                                  num_cores=NC, num_subcores=NS)
    return pl.kernel(
        _edge_pass,
        out_type=jax.ShapeDtypeStruct((NC, N_PAD), jnp.float32),
        mesh=mesh,
        scratch_types=[
            pltpu.VMEM_SHARED((N_PAD,), jnp.float32),
            pltpu.VMEM((N_PAD,), jnp.float32),
            pltpu.VMEM((G, 128), jnp.int32),
            pltpu.VMEM((G, 128), jnp.float32),
            pltpu.VMEM((G, 128), jnp.float32),
            pltpu.VMEM((G, 128), jnp.int32),
            pltpu.VMEM((CHUNK,), jnp.float32),
            pltpu.SemaphoreType.DMA,
        ],
        compiler_params=pltpu.CompilerParams(needs_layout_passes=False),
        interpret=interpret,
    )


def _node_stage(w_ref, s_ref, x_ref, t_ref, r_ref):
    sv = s_ref[...]
    xv = x_ref[...]
    t = jnp.zeros_like(sv)
    r = jnp.zeros_like(sv)
    for k in range(H):
        h = jnp.maximum(sv * w_ref[0, k] + xv * w_ref[2, k] + w_ref[1, k], 0.0)
        t = t + h * w_ref[3, k]
        r = r + h * w_ref[4, k]
    t_ref[...] = t
    r_ref[...] = r + w_ref[5, 0]


_ROWS = N_PAD // 128
_BLK = 80


def _node_kernel(wmat, s2d, x2d, interpret=False):
    return pl.pallas_call(
        _node_stage,
        grid=(_ROWS // _BLK,),
        in_specs=[
            pl.BlockSpec(memory_space=pltpu.SMEM),
            pl.BlockSpec((_BLK, 128), lambda i: (i, 0)),
            pl.BlockSpec((_BLK, 128), lambda i: (i, 0)),
        ],
        out_specs=[
            pl.BlockSpec((_BLK, 128), lambda i: (i, 0)),
            pl.BlockSpec((_BLK, 128), lambda i: (i, 0)),
        ],
        out_shape=[
            jax.ShapeDtypeStruct((_ROWS, 128), jnp.float32),
            jax.ShapeDtypeStruct((_ROWS, 128), jnp.float32),
        ],
        interpret=interpret,
    )(wmat, s2d, x2d)


@jax.jit
def _run(x, edge_index, edge_weight,
         W1_rel, b1_rel, W1_root, W2_rel, b2_rel, W2_root):
    src = edge_index[0]
    dst = edge_index[1]
    pad = E_PAD - E
    src_p = jnp.concatenate([src, jnp.zeros((pad,), jnp.int32)])
    dst_p = jnp.concatenate([dst, jnp.full((pad,), N, jnp.int32)])
    w_p = jnp.concatenate([edge_weight, jnp.zeros((pad,), jnp.float32)])
    src2d = src_p.reshape(E_PAD // 128, 128)
    dst2d = dst_p.reshape(E_PAD // 128, 128)
    w2d = w_p.reshape(E_PAD // 128, 128)

    x_flat = x.reshape(-1)
    x_pad = jnp.concatenate([x_flat, jnp.zeros((N_PAD - N,), jnp.float32)])

    edge_pass = _make_edge_pass()

    s_part = edge_pass(x_pad, src2d, w2d, dst2d)
    s_pad = s_part[0] + s_part[1]

    wmat = jnp.stack([
        W1_rel[:, 0], b1_rel, W1_root[:, 0],
        W2_rel[0, :], W2_root[0, :],
        jnp.full((H,), b2_rel[0], jnp.float32),
    ])
    t2d, r2d = _node_kernel(wmat, s_pad.reshape(_ROWS, 128),
                            x_pad.reshape(_ROWS, 128))
    t_pad = t2d.reshape(-1)

    u_part = edge_pass(t_pad, src2d, w2d, dst2d)
    out_pad = u_part[0] + u_part[1] + r2d.reshape(-1)
    return out_pad[:N]


def kernel(x, edge_index, edge_weight,
           W1_rel, b1_rel, W1_root, W2_rel, b2_rel, W2_root):
    return _run(x, edge_index, edge_weight,
                W1_rel, b1_rel, W1_root, W2_rel, b2_rel, W2_root)


# speedup vs baseline: 165.3468x; 1.3903x over previous
import jax
import jax.numpy as jnp
from jax import lax
from jax.experimental import pallas as pl
from jax.experimental.pallas import tpu as pltpu
from jax.experimental.pallas import tpu_sc as plsc

N = 100000
E = 3200000
H = 16

NC = 2
NS = 16
L = 16

N_PAD = 100352
CHUNK = 2048
ROWS_PER_CHUNK = CHUNK // 128
N_TILES = NC * NS
CHUNKS_PER_TILE = 50
E_PAD = N_TILES * CHUNKS_PER_TILE * CHUNK
E_ROWS = E_PAD // 128
ACC_SLICE = N_PAD // NS
ZV = 2048


def _edge_pass(table_hbm, src_hbm, w_hbm, dst_hbm, out_hbm,
               acc_sh, table_v, src0, src1, w0, w1, dst0, dst1, m0, m1,
               zv, sem_t, si0, si1, ss0, ss1):
    c = lax.axis_index("c")
    s = lax.axis_index("s")
    wid = c * NS + s
    chunk0 = wid * CHUNKS_PER_TILE

    srcs = (src0, src1)
    ws = (w0, w1)
    dsts = (dst0, dst1)
    ms = (m0, m1)
    sis = (si0, si1)
    sss = (ss0, ss1)

    def in_start(chunk_idx, p):
        row = (chunk0 + chunk_idx) * ROWS_PER_CHUNK
        sl = pl.ds(row, ROWS_PER_CHUNK)
        pltpu.async_copy(src_hbm.at[sl], srcs[p], sis[p])
        pltpu.async_copy(w_hbm.at[sl], ws[p], sis[p])
        pltpu.async_copy(dst_hbm.at[sl], dsts[p], sis[p])

    def in_wait(chunk_idx, p):
        row = (chunk0 + chunk_idx) * ROWS_PER_CHUNK
        sl = pl.ds(row, ROWS_PER_CHUNK)
        pltpu.make_async_copy(src_hbm.at[sl], srcs[p], sis[p]).wait()
        pltpu.make_async_copy(w_hbm.at[sl], ws[p], sis[p]).wait()
        pltpu.make_async_copy(dst_hbm.at[sl], dsts[p], sis[p]).wait()

    table_cp = pltpu.async_copy(table_hbm, table_v, sem_t)
    in_start(0, 0)
    in_start(1, 1)

    def zero_body(i, _):
        zv[pl.ds(i * L, L)] = jnp.zeros((L,), jnp.float32)
        return 0
    lax.fori_loop(0, ZV // L, zero_body, 0)
    base = s * ACC_SLICE
    for q in range(3):
        pltpu.sync_copy(zv, acc_sh.at[pl.ds(base + q * ZV, ZV)])
    pltpu.sync_copy(zv.at[pl.ds(0, ACC_SLICE - 3 * ZV)],
                    acc_sh.at[pl.ds(base + 3 * ZV, ACC_SLICE - 3 * ZV)])
    table_cp.wait()
    plsc.subcore_barrier()

    def gather(p):
        src_v, w_v, m_v = srcs[p], ws[p], ms[p]

        def g_body(j, _):
            for k in range(128 // L):
                idx = src_v[j, pl.ds(k * L, L)]
                vals = plsc.load_gather(table_v, [idx])
                m_v[j, pl.ds(k * L, L)] = vals * w_v[j, pl.ds(k * L, L)]
            return 0
        lax.fori_loop(0, ROWS_PER_CHUNK, g_body, 0)

    def scat_start(p):
        return [pltpu.async_copy(ms[p].at[j], acc_sh.at[dsts[p].at[j]],
                                 sss[p], add=True)
                for j in range(ROWS_PER_CHUNK)]

    def pair_body(p, _):
        a = 2 * p
        in_wait(a, 0)
        gather(0)
        h0 = scat_start(0)
        in_wait(a + 1, 1)
        gather(1)
        h1 = scat_start(1)
        for h in h0:
            h.wait()

        @pl.when(p < CHUNKS_PER_TILE // 2 - 1)
        def _():
            in_start(a + 2, 0)
        for h in h1:
            h.wait()

        @pl.when(p < CHUNKS_PER_TILE // 2 - 1)
        def _():
            in_start(a + 3, 1)
        return 0

    lax.fori_loop(0, CHUNKS_PER_TILE // 2, pair_body, 0)
    plsc.subcore_barrier()

    pltpu.sync_copy(acc_sh.at[pl.ds(base, ACC_SLICE)],
                    out_hbm.at[c, pl.ds(base, ACC_SLICE)])


def _make_edge_pass(interpret=False):
    mesh = plsc.VectorSubcoreMesh(core_axis_name="c", subcore_axis_name="s",
                                  num_cores=NC, num_subcores=NS)
    r = ROWS_PER_CHUNK
    return pl.kernel(
        _edge_pass,
        out_type=jax.ShapeDtypeStruct((NC, N_PAD), jnp.float32),
        mesh=mesh,
        scratch_types=[
            pltpu.VMEM_SHARED((N_PAD,), jnp.float32),
            pltpu.VMEM((N_PAD,), jnp.float32),
            pltpu.VMEM((r, 128), jnp.int32),
            pltpu.VMEM((r, 128), jnp.int32),
            pltpu.VMEM((r, 128), jnp.float32),
            pltpu.VMEM((r, 128), jnp.float32),
            pltpu.VMEM((r, 128), jnp.int32),
            pltpu.VMEM((r, 128), jnp.int32),
            pltpu.VMEM((r, 128), jnp.float32),
            pltpu.VMEM((r, 128), jnp.float32),
            pltpu.VMEM((ZV,), jnp.float32),
            pltpu.SemaphoreType.DMA,
            pltpu.SemaphoreType.DMA,
            pltpu.SemaphoreType.DMA,
            pltpu.SemaphoreType.DMA,
            pltpu.SemaphoreType.DMA,
        ],
        compiler_params=pltpu.CompilerParams(needs_layout_passes=False),
        interpret=interpret,
    )


def _node_stage(w_ref, s_ref, x_ref, t_ref, r_ref):
    sv = s_ref[...]
    xv = x_ref[...]
    t = jnp.zeros_like(sv)
    r = jnp.zeros_like(sv)
    for k in range(H):
        h = jnp.maximum(sv * w_ref[0, k] + xv * w_ref[2, k] + w_ref[1, k], 0.0)
        t = t + h * w_ref[3, k]
        r = r + h * w_ref[4, k]
    t_ref[...] = t
    r_ref[...] = r + w_ref[5, 0]


_ROWS = N_PAD // 128
_BLK = 112


def _node_kernel(wmat, s2d, x2d, interpret=False):
    return pl.pallas_call(
        _node_stage,
        grid=(_ROWS // _BLK,),
        in_specs=[
            pl.BlockSpec(memory_space=pltpu.SMEM),
            pl.BlockSpec((_BLK, 128), lambda i: (i, 0)),
            pl.BlockSpec((_BLK, 128), lambda i: (i, 0)),
        ],
        out_specs=[
            pl.BlockSpec((_BLK, 128), lambda i: (i, 0)),
            pl.BlockSpec((_BLK, 128), lambda i: (i, 0)),
        ],
        out_shape=[
            jax.ShapeDtypeStruct((_ROWS, 128), jnp.float32),
            jax.ShapeDtypeStruct((_ROWS, 128), jnp.float32),
        ],
        interpret=interpret,
    )(wmat, s2d, x2d)


@jax.jit
def _run(x, edge_index, edge_weight,
         W1_rel, b1_rel, W1_root, W2_rel, b2_rel, W2_root):
    src = edge_index[0]
    dst = edge_index[1]
    pad = E_PAD - E
    src_p = jnp.concatenate([src, jnp.zeros((pad,), jnp.int32)])
    dst_p = jnp.concatenate([dst, jnp.full((pad,), N, jnp.int32)])
    w_p = jnp.concatenate([edge_weight, jnp.zeros((pad,), jnp.float32)])
    src2d = src_p.reshape(E_ROWS, 128)
    dst2d = dst_p.reshape(E_ROWS, 128)
    w2d = w_p.reshape(E_ROWS, 128)

    x_flat = x.reshape(-1)
    x_pad = jnp.concatenate([x_flat, jnp.zeros((N_PAD - N,), jnp.float32)])

    edge_pass = _make_edge_pass()

    s_part = edge_pass(x_pad, src2d, w2d, dst2d)
    s_pad = s_part[0] + s_part[1]

    wmat = jnp.stack([
        W1_rel[:, 0], b1_rel, W1_root[:, 0],
        W2_rel[0, :], W2_root[0, :],
        jnp.full((H,), b2_rel[0], jnp.float32),
    ])
    t2d, r2d = _node_kernel(wmat, s_pad.reshape(_ROWS, 128),
                            x_pad.reshape(_ROWS, 128))
    t_pad = t2d.reshape(-1)

    u_part = edge_pass(t_pad, src2d, w2d, dst2d)
    out_pad = u_part[0] + u_part[1] + r2d.reshape(-1)
    return out_pad[:N]


def kernel(x, edge_index, edge_weight,
           W1_rel, b1_rel, W1_root, W2_rel, b2_rel, W2_root):
    return _run(x, edge_index, edge_weight,
                W1_rel, b1_rel, W1_root, W2_rel, b2_rel, W2_root)
